# baseline (speedup 1.0000x reference)
import functools

import jax
import jax.numpy as jnp
from jax import lax
from jax.experimental import pallas as pl
from jax.experimental.pallas import tpu as pltpu
from jax.experimental.pallas import tpu_sc as plsc

BATCH = 16384
DIM = 32
NUM_WORKERS = 32
B_PER_W = BATCH // NUM_WORKERS
N_CHUNKS = 4
CHUNK = B_PER_W // N_CHUNKS
GROUPS = B_PER_W // 16


def _pda_body(uidx_hbm, iidx_hbm, utab_hbm, itab_hbm, out_hbm,
              idx_u, idx_i, rows_u, rows_i, out_v, sem):
    wid = lax.axis_index("s") * 2 + lax.axis_index("c")

    pltpu.sync_copy(uidx_hbm.at[wid], idx_u)
    pltpu.sync_copy(iidx_hbm.at[wid], idx_i)

    copies = []
    for j in range(N_CHUNKS):
        sl = pl.ds(j * CHUNK, CHUNK)
        copies.append(pltpu.async_copy(utab_hbm.at[idx_u.at[j]], rows_u.at[sl], sem))
        copies.append(pltpu.async_copy(itab_hbm.at[idx_i.at[j]], rows_i.at[sl], sem))
    for c in copies:
        c.wait()

    lane = lax.iota(jnp.int32, 16)

    def group(g, _):
        row_vec = g * 16 + lane
        acc = None
        for d in range(DIM):
            dsplat = jnp.full((16,), d, jnp.int32)
            u = plsc.load_gather(rows_u, [row_vec, dsplat])
            v = plsc.load_gather(rows_i, [row_vec, dsplat])
            acc = u * v if acc is None else acc + u * v
        res = 1.0 / (1.0 + jnp.exp(-acc))
        out_v[pl.ds(g * 16, 16)] = res
        return _

    lax.fori_loop(0, GROUPS, group, None)

    pltpu.sync_copy(out_v, out_hbm.at[wid])


@functools.partial(jax.jit, static_argnames=())
def _pda(uidx, iidx, utab, itab):
    mesh = plsc.VectorSubcoreMesh(core_axis_name="c", subcore_axis_name="s")
    f = pl.kernel(
        _pda_body,
        mesh=mesh,
        compiler_params=pltpu.CompilerParams(needs_layout_passes=False, use_tc_tiling_on_sc=False),
        out_type=jax.ShapeDtypeStruct((NUM_WORKERS, B_PER_W), jnp.float32),
        scratch_types=[
            pltpu.VMEM((N_CHUNKS, CHUNK), jnp.int32),
            pltpu.VMEM((N_CHUNKS, CHUNK), jnp.int32),
            pltpu.VMEM((B_PER_W, DIM), jnp.float32),
            pltpu.VMEM((B_PER_W, DIM), jnp.float32),
            pltpu.VMEM((B_PER_W,), jnp.float32),
            pltpu.SemaphoreType.DMA,
        ],
    )
    return f(uidx, iidx, utab, itab)


def kernel(userIdx, itemIdx, uEmbed, iEmbed):
    uidx = userIdx.astype(jnp.int32).reshape(NUM_WORKERS, N_CHUNKS, CHUNK)
    iidx = itemIdx.astype(jnp.int32).reshape(NUM_WORKERS, N_CHUNKS, CHUNK)
    out = _pda(uidx, iidx, uEmbed, iEmbed)
    return out.reshape(-1)


# speedup vs baseline: 1.4969x; 1.4969x over previous
import jax
import jax.numpy as jnp
from jax import lax
from jax.experimental import pallas as pl
from jax.experimental.pallas import tpu as pltpu
from jax.experimental.pallas import tpu_sc as plsc

BATCH = 16384
DIM = 32
NUM_WORKERS = 32
B_PER_W = BATCH // NUM_WORKERS
N_PHASES = 2
PHASE = B_PER_W // N_PHASES
GROUPS = PHASE // 16


def _pda_body(uidx_hbm, iidx_hbm, utab_hbm, itab_hbm, out_hbm,
              idx_uv, idx_iv, rows_u, rows_i, out_v, sem):
    wid = lax.axis_index("s") * 2 + lax.axis_index("c")
    base = wid * B_PER_W

    pltpu.sync_copy(uidx_hbm.at[pl.ds(base, B_PER_W)], idx_uv)
    pltpu.sync_copy(iidx_hbm.at[pl.ds(base, B_PER_W)], idx_iv)

    lane = lax.iota(jnp.int32, 16)

    def phase_step(p, _):
        pbase = p * PHASE

        def fire(v, _):
            uvec = idx_uv[pl.ds(pbase + v * 16, 16)]
            ivec = idx_iv[pl.ds(pbase + v * 16, 16)]
            for l in range(16):
                pltpu.async_copy(utab_hbm.at[uvec[l]],
                                 rows_u.at[v * 16 + l], sem)
                pltpu.async_copy(itab_hbm.at[ivec[l]],
                                 rows_i.at[v * 16 + l], sem)
            return _

        lax.fori_loop(0, PHASE // 16, fire, None)

        pltpu.make_async_copy(utab_hbm.at[pl.ds(0, PHASE)], rows_u, sem).wait()
        pltpu.make_async_copy(itab_hbm.at[pl.ds(0, PHASE)], rows_i, sem).wait()

        def group(g, _):
            row_vec = g * 16 + lane
            acc = None
            for d in range(DIM):
                dsplat = jnp.full((16,), d, jnp.int32)
                u = plsc.load_gather(rows_u, [row_vec, dsplat])
                v = plsc.load_gather(rows_i, [row_vec, dsplat])
                acc = u * v if acc is None else acc + u * v
            res = 1.0 / (1.0 + jnp.exp(-acc))
            out_v[pl.ds(pbase + g * 16, 16)] = res
            return _

        lax.fori_loop(0, GROUPS, group, None)
        return _

    lax.fori_loop(0, N_PHASES, phase_step, None)

    pltpu.sync_copy(out_v, out_hbm.at[pl.ds(base, B_PER_W)])


@jax.jit
def _pda(uidx, iidx, utab, itab):
    mesh = plsc.VectorSubcoreMesh(core_axis_name="c", subcore_axis_name="s")
    f = pl.kernel(
        _pda_body,
        mesh=mesh,
        compiler_params=pltpu.CompilerParams(needs_layout_passes=False),
        out_type=jax.ShapeDtypeStruct((BATCH,), jnp.float32),
        scratch_types=[
            pltpu.VMEM((B_PER_W,), jnp.int32),
            pltpu.VMEM((B_PER_W,), jnp.int32),
            pltpu.VMEM((PHASE, DIM), jnp.float32),
            pltpu.VMEM((PHASE, DIM), jnp.float32),
            pltpu.VMEM((B_PER_W,), jnp.float32),
            pltpu.SemaphoreType.DMA,
        ],
    )
    return f(uidx, iidx, utab, itab)


def kernel(userIdx, itemIdx, uEmbed, iEmbed):
    return _pda(userIdx.astype(jnp.int32), itemIdx.astype(jnp.int32),
                uEmbed, iEmbed)
